# Initial kernel scaffold; baseline (speedup 1.0000x reference)
import jax
import jax.numpy as jnp
from jax.experimental import pallas as pl


def kernel(x, edge_index, W1, b1, W2, b2, Wc, bc):
    raise NotImplementedError("write your pallas kernel here")


# speedup vs baseline: 15.3394x; 15.3394x over previous
import functools

import jax
import jax.numpy as jnp
from jax import lax
from jax.experimental import pallas as pl
from jax.experimental.pallas import tpu as pltpu
from jax.experimental.pallas import tpu_sc as plsc

N = 10000
NPAD = 10240
NB = 1024
E = 320000
NC = 2
NS = 16
CH = 80
GRP = 25
RPT = NPAD // NS
F32 = jnp.float32


_MESH = dict(core_axis_name="c", subcore_axis_name="s")
_SC_PARAMS = pltpu.CompilerParams(needs_layout_passes=False)
_EPT_DEG = E // (NC * NS)


def _deg_body(dst_hbm, out_hbm, idx_v, hist_v):
    c = lax.axis_index("c")
    s = lax.axis_index("s")
    wid = c * NS + s
    pltpu.sync_copy(dst_hbm.at[wid], idx_v)

    def zero(i, carry):
        hist_v[0, pl.ds(i * 16, 16)] = jnp.zeros((16,), F32)
        return carry

    lax.fori_loop(0, NPAD // 16, zero, 0)
    ones = jnp.ones((16,), F32)
    row0 = jnp.zeros((16,), jnp.int32)

    def acc(i, carry):
        idx = idx_v[0, pl.ds(i * 16, 16)]
        plsc.addupdate_scatter(hist_v, [row0, idx], ones)
        return carry

    lax.fori_loop(0, _EPT_DEG // 16, acc, 0)
    pltpu.sync_copy(hist_v, out_hbm.at[wid])


def _deg(dst32):
    k = pl.kernel(
        _deg_body,
        out_type=jax.ShapeDtypeStruct((NC * NS, 1, NPAD), F32),
        mesh=plsc.VectorSubcoreMesh(**_MESH),
        compiler_params=_SC_PARAMS,
        scratch_types=[
            pltpu.VMEM((1, _EPT_DEG), jnp.int32),
            pltpu.VMEM((1, NPAD), F32),
        ],
    )
    return k(dst32.reshape(NC * NS, 1, _EPT_DEG)).reshape(NC * NS, NPAD)


def _make_prop(n_chunks, table_rows):
    def body(table, src_hbm, dst_hbm, out_hbm, src_v, dst_v, gbuf, zb, acc_sh, sem):
        c = lax.axis_index("c")
        s = lax.axis_index("s")

        def zrow(r, carry):
            def zcol(k, carry2):
                zb[r, pl.ds(k * 16, 16)] = jnp.zeros((16,), F32)
                return carry2

            return lax.fori_loop(0, 8, zcol, carry)

        lax.fori_loop(0, 128, zrow, 0)

        def zcp(j, carry):
            pltpu.sync_copy(zb, acc_sh.at[pl.ds(s * RPT + j * 128, 128)])
            return carry

        lax.fori_loop(0, RPT // 128, zcp, 0)
        plsc.subcore_barrier()

        def group(g, carry):
            pltpu.sync_copy(src_hbm.at[c, s, g], src_v)
            pltpu.sync_copy(dst_hbm.at[c, s, g], dst_v)

            def step(j, carry2):
                pltpu.async_copy(table.at[src_v.at[j]], gbuf, sem).wait()
                pltpu.sync_copy(gbuf, acc_sh.at[dst_v.at[j]], add=True)
                return carry2

            return lax.fori_loop(0, GRP, step, carry)

        lax.fori_loop(0, n_chunks // GRP, group, 0)
        plsc.subcore_barrier()

        def wb(j, carry):
            base = s * RPT + j * 128
            pltpu.sync_copy(acc_sh.at[pl.ds(base, 128)], zb)
            pltpu.sync_copy(zb, out_hbm.at[c, pl.ds(base, 128)])
            return carry

        lax.fori_loop(0, RPT // 128, wb, 0)

    return pl.kernel(
        body,
        out_type=jax.ShapeDtypeStruct((NC, NPAD, 128), F32),
        mesh=plsc.VectorSubcoreMesh(**_MESH),
        compiler_params=_SC_PARAMS,
        scratch_types=[
            pltpu.VMEM((GRP, CH), jnp.int32),
            pltpu.VMEM((GRP, CH), jnp.int32),
            pltpu.VMEM((CH, 128), F32),
            pltpu.VMEM((128, 128), F32),
            pltpu.VMEM_SHARED((NPAD, 128), F32),
            pltpu.SemaphoreType.DMA,
        ],
    )


def _dinv_of(dT_blk):
    return lax.rsqrt(jnp.sum(dT_blk, axis=1, keepdims=True) + 1.0)


def _tc1_body(x_ref, w_ref, dT_ref, y_ref):
    dinv = _dinv_of(dT_ref[...])
    xw = jnp.dot(x_ref[...], w_ref[...], preferred_element_type=F32)
    y = xw * dinv
    y_ref[0] = y[:, :128]
    y_ref[1] = y[:, 128:]


def _tc1(x_pad, W1, degT, interpret=False):
    return pl.pallas_call(
        _tc1_body,
        grid=(NPAD // NB,),
        in_specs=[
            pl.BlockSpec((NB, 128), lambda i: (i, 0)),
            pl.BlockSpec((128, 256), lambda i: (0, 0)),
            pl.BlockSpec((NB, 32), lambda i: (i, 0)),
        ],
        out_specs=pl.BlockSpec((2, NB, 128), lambda i: (0, i, 0)),
        out_shape=jax.ShapeDtypeStruct((2, NPAD, 128), F32),
        interpret=interpret,
    )(x_pad, W1, degT)


def _tc2_body(a_ref, y_ref, dT_ref, w2_ref, b1_ref, o_ref):
    dinv = _dinv_of(dT_ref[...])
    h = jnp.concatenate([a_ref[0] + y_ref[0], a_ref[1] + y_ref[1]], axis=1)
    h = jnp.maximum(h * dinv + b1_ref[...], 0.0)
    o_ref[...] = jnp.dot(h, w2_ref[...], preferred_element_type=F32) * dinv


def _tc2(acc1, y1, degT, W2, b1r, interpret=False):
    return pl.pallas_call(
        _tc2_body,
        grid=(NPAD // NB,),
        in_specs=[
            pl.BlockSpec((2, NB, 128), lambda i: (0, i, 0)),
            pl.BlockSpec((2, NB, 128), lambda i: (0, i, 0)),
            pl.BlockSpec((NB, 32), lambda i: (i, 0)),
            pl.BlockSpec((256, 128), lambda i: (0, 0)),
            pl.BlockSpec((1, 256), lambda i: (0, 0)),
        ],
        out_specs=pl.BlockSpec((NB, 128), lambda i: (i, 0)),
        out_shape=jax.ShapeDtypeStruct((NPAD, 128), F32),
        interpret=interpret,
    )(acc1, y1, degT, W2, b1r)


def _tc3_body(a_ref, y_ref, dT_ref, b2_ref, wc_ref, bc_ref, o_ref):
    dinv = _dinv_of(dT_ref[...])
    h = jnp.maximum((a_ref[0] + a_ref[1] + y_ref[...]) * dinv + b2_ref[...], 0.0)
    logits = jnp.dot(h, wc_ref[...], preferred_element_type=F32) + bc_ref[...]
    m = jnp.max(logits, axis=1, keepdims=True)
    lse = jnp.log(jnp.sum(jnp.exp(logits - m), axis=1, keepdims=True))
    o_ref[...] = logits - m - lse


def _tc3(acc2, y2, degT, b2r, Wc, bcr, interpret=False):
    return pl.pallas_call(
        _tc3_body,
        grid=(NPAD // NB,),
        in_specs=[
            pl.BlockSpec((2, NB, 128), lambda i: (0, i, 0)),
            pl.BlockSpec((NB, 128), lambda i: (i, 0)),
            pl.BlockSpec((NB, 32), lambda i: (i, 0)),
            pl.BlockSpec((1, 128), lambda i: (0, 0)),
            pl.BlockSpec((128, 10), lambda i: (0, 0)),
            pl.BlockSpec((1, 10), lambda i: (0, 0)),
        ],
        out_specs=pl.BlockSpec((NB, 10), lambda i: (i, 0)),
        out_shape=jax.ShapeDtypeStruct((NPAD, 10), F32),
        interpret=interpret,
    )(acc2, y2, degT, b2r, Wc, bcr)


def kernel(x, edge_index, W1, b1, W2, b2, Wc, bc):
    ei = edge_index.astype(jnp.int32)
    src = ei[0]
    dst = ei[1]
    x_pad = jnp.pad(x, ((0, NPAD - N), (0, 0)))

    deg32 = _deg(dst)
    degT = deg32.T

    y1 = _tc1(x_pad, W1, degT)

    nch1 = E // (NS * CH)
    src1 = jnp.stack([src, src + NPAD]).reshape(NC, NS, nch1 // GRP, GRP, CH)
    dst1 = jnp.stack([dst, dst]).reshape(NC, NS, nch1 // GRP, GRP, CH)
    acc1 = _make_prop(nch1, 2 * NPAD)(
        y1.reshape(2 * NPAD, 128), src1, dst1
    )

    y2 = _tc2(acc1, y1, degT, W2, b1.reshape(1, 256))

    nch2 = E // (NC * NS * CH)
    src2 = src.reshape(NC, NS, nch2 // GRP, GRP, CH)
    dst2 = dst.reshape(NC, NS, nch2 // GRP, GRP, CH)
    acc2 = _make_prop(nch2, NPAD)(y2, src2, dst2)

    out = _tc3(acc2, y2, degT, b2.reshape(1, 128), Wc, bc.reshape(1, 10))
    return out[:N]


# speedup vs baseline: 20.8388x; 1.3585x over previous
import functools

import jax
import jax.numpy as jnp
from jax import lax
from jax.experimental import pallas as pl
from jax.experimental.pallas import tpu as pltpu
from jax.experimental.pallas import tpu_sc as plsc

N = 10000
NPAD = 10240
NB = 1024
E = 320000
NC = 2
NS = 16
CH = 100
GRP = 20
ZB = 64
RPT = NPAD // NS
F32 = jnp.float32


_MESH = dict(core_axis_name="c", subcore_axis_name="s")
_SC_PARAMS = pltpu.CompilerParams(needs_layout_passes=False)
_EPT_DEG = E // (NC * NS)


def _deg_body(dst_hbm, out_hbm, idx_v, hist_v):
    c = lax.axis_index("c")
    s = lax.axis_index("s")
    wid = c * NS + s
    pltpu.sync_copy(dst_hbm.at[wid], idx_v)

    def zero(i, carry):
        hist_v[0, pl.ds(i * 16, 16)] = jnp.zeros((16,), F32)
        return carry

    lax.fori_loop(0, NPAD // 16, zero, 0)
    ones = jnp.ones((16,), F32)
    row0 = jnp.zeros((16,), jnp.int32)

    def acc(i, carry):
        idx = idx_v[0, pl.ds(i * 16, 16)]
        plsc.addupdate_scatter(hist_v, [row0, idx], ones)
        return carry

    lax.fori_loop(0, _EPT_DEG // 16, acc, 0)
    pltpu.sync_copy(hist_v, out_hbm.at[wid])


def _deg(dst32):
    k = pl.kernel(
        _deg_body,
        out_type=jax.ShapeDtypeStruct((NC * NS, 1, NPAD), F32),
        mesh=plsc.VectorSubcoreMesh(**_MESH),
        compiler_params=_SC_PARAMS,
        scratch_types=[
            pltpu.VMEM((1, _EPT_DEG), jnp.int32),
            pltpu.VMEM((1, NPAD), F32),
        ],
    )
    return k(dst32.reshape(NC * NS, 1, _EPT_DEG)).reshape(NC * NS, NPAD)


def _make_prop(n_chunks, table_rows):
    def body(table, src_hbm, dst_hbm, out_hbm, src_v, dst_v, g0, g1, zb,
             acc_sh, sem0, sem1):
        c = lax.axis_index("c")
        s = lax.axis_index("s")

        def zrow(r, carry):
            def zcol(k, carry2):
                zb[r, pl.ds(k * 16, 16)] = jnp.zeros((16,), F32)
                return carry2

            return lax.fori_loop(0, 8, zcol, carry)

        lax.fori_loop(0, ZB, zrow, 0)

        def zcp(j, carry):
            pltpu.sync_copy(zb, acc_sh.at[pl.ds(s * RPT + j * ZB, ZB)])
            return carry

        lax.fori_loop(0, RPT // ZB, zcp, 0)
        plsc.subcore_barrier()

        def do_pair(j0, prefetch):
            j1 = j0 + 1
            pltpu.make_async_copy(table.at[src_v.at[j0]], g0, sem0).wait()
            pltpu.async_copy(table.at[src_v.at[j1]], g1, sem1)
            pltpu.sync_copy(g0, acc_sh.at[dst_v.at[j0]], add=True)
            pltpu.make_async_copy(table.at[src_v.at[j1]], g1, sem1).wait()
            if prefetch:
                pltpu.async_copy(table.at[src_v.at[j0 + 2]], g0, sem0)
            pltpu.sync_copy(g1, acc_sh.at[dst_v.at[j1]], add=True)

        def group(g, carry):
            pltpu.sync_copy(src_hbm.at[c, s, g], src_v)
            pltpu.sync_copy(dst_hbm.at[c, s, g], dst_v)
            pltpu.async_copy(table.at[src_v.at[0]], g0, sem0)

            def pair_loop(p, carry2):
                do_pair(p * 2, True)
                return carry2

            lax.fori_loop(0, GRP // 2 - 1, pair_loop, carry)
            do_pair(GRP - 2, False)
            return carry

        lax.fori_loop(0, n_chunks // GRP, group, 0)
        plsc.subcore_barrier()

        def wb(j, carry):
            base = s * RPT + j * ZB
            pltpu.sync_copy(acc_sh.at[pl.ds(base, ZB)], zb)
            pltpu.sync_copy(zb, out_hbm.at[c, pl.ds(base, ZB)])
            return carry

        lax.fori_loop(0, RPT // ZB, wb, 0)

    return pl.kernel(
        body,
        out_type=jax.ShapeDtypeStruct((NC, NPAD, 128), F32),
        mesh=plsc.VectorSubcoreMesh(**_MESH),
        compiler_params=_SC_PARAMS,
        scratch_types=[
            pltpu.VMEM((GRP, CH), jnp.int32),
            pltpu.VMEM((GRP, CH), jnp.int32),
            pltpu.VMEM((CH, 128), F32),
            pltpu.VMEM((CH, 128), F32),
            pltpu.VMEM((ZB, 128), F32),
            pltpu.VMEM_SHARED((NPAD, 128), F32),
            pltpu.SemaphoreType.DMA,
            pltpu.SemaphoreType.DMA,
        ],
    )


def _dinv_of(dT_blk):
    return lax.rsqrt(jnp.sum(dT_blk, axis=1, keepdims=True) + 1.0)


def _tc1_body(x_ref, w_ref, dT_ref, y_ref):
    dinv = _dinv_of(dT_ref[...])
    xw = jnp.dot(x_ref[...], w_ref[...], preferred_element_type=F32)
    y = xw * dinv
    y_ref[0] = y[:, :128]
    y_ref[1] = y[:, 128:]


def _tc1(x_pad, W1, degT, interpret=False):
    return pl.pallas_call(
        _tc1_body,
        grid=(NPAD // NB,),
        in_specs=[
            pl.BlockSpec((NB, 128), lambda i: (i, 0)),
            pl.BlockSpec((128, 256), lambda i: (0, 0)),
            pl.BlockSpec((NB, 32), lambda i: (i, 0)),
        ],
        out_specs=pl.BlockSpec((2, NB, 128), lambda i: (0, i, 0)),
        out_shape=jax.ShapeDtypeStruct((2, NPAD, 128), F32),
        interpret=interpret,
    )(x_pad, W1, degT)


def _tc2_body(a_ref, y_ref, dT_ref, w2_ref, b1_ref, o_ref):
    dinv = _dinv_of(dT_ref[...])
    h = jnp.concatenate([a_ref[0] + y_ref[0], a_ref[1] + y_ref[1]], axis=1)
    h = jnp.maximum(h * dinv + b1_ref[...], 0.0)
    o_ref[...] = jnp.dot(h, w2_ref[...], preferred_element_type=F32) * dinv


def _tc2(acc1, y1, degT, W2, b1r, interpret=False):
    return pl.pallas_call(
        _tc2_body,
        grid=(NPAD // NB,),
        in_specs=[
            pl.BlockSpec((2, NB, 128), lambda i: (0, i, 0)),
            pl.BlockSpec((2, NB, 128), lambda i: (0, i, 0)),
            pl.BlockSpec((NB, 32), lambda i: (i, 0)),
            pl.BlockSpec((256, 128), lambda i: (0, 0)),
            pl.BlockSpec((1, 256), lambda i: (0, 0)),
        ],
        out_specs=pl.BlockSpec((NB, 128), lambda i: (i, 0)),
        out_shape=jax.ShapeDtypeStruct((NPAD, 128), F32),
        interpret=interpret,
    )(acc1, y1, degT, W2, b1r)


def _tc3_body(a_ref, y_ref, dT_ref, b2_ref, wc_ref, bc_ref, o_ref):
    dinv = _dinv_of(dT_ref[...])
    h = jnp.maximum((a_ref[0] + a_ref[1] + y_ref[...]) * dinv + b2_ref[...], 0.0)
    logits = jnp.dot(h, wc_ref[...], preferred_element_type=F32) + bc_ref[...]
    m = jnp.max(logits, axis=1, keepdims=True)
    lse = jnp.log(jnp.sum(jnp.exp(logits - m), axis=1, keepdims=True))
    o_ref[...] = logits - m - lse


def _tc3(acc2, y2, degT, b2r, Wc, bcr, interpret=False):
    return pl.pallas_call(
        _tc3_body,
        grid=(NPAD // NB,),
        in_specs=[
            pl.BlockSpec((2, NB, 128), lambda i: (0, i, 0)),
            pl.BlockSpec((NB, 128), lambda i: (i, 0)),
            pl.BlockSpec((NB, 32), lambda i: (i, 0)),
            pl.BlockSpec((1, 128), lambda i: (0, 0)),
            pl.BlockSpec((128, 10), lambda i: (0, 0)),
            pl.BlockSpec((1, 10), lambda i: (0, 0)),
        ],
        out_specs=pl.BlockSpec((NB, 10), lambda i: (i, 0)),
        out_shape=jax.ShapeDtypeStruct((NPAD, 10), F32),
        interpret=interpret,
    )(acc2, y2, degT, b2r, Wc, bcr)


def kernel(x, edge_index, W1, b1, W2, b2, Wc, bc):
    ei = edge_index.astype(jnp.int32)
    src = ei[0]
    dst = ei[1]
    x_pad = jnp.pad(x, ((0, NPAD - N), (0, 0)))

    deg32 = _deg(dst)
    degT = deg32.T

    y1 = _tc1(x_pad, W1, degT)

    nch1 = E // (NS * CH)
    src1 = jnp.stack([src, src + NPAD]).reshape(NC, NS, nch1 // GRP, GRP, CH)
    dst1 = jnp.stack([dst, dst]).reshape(NC, NS, nch1 // GRP, GRP, CH)
    acc1 = _make_prop(nch1, 2 * NPAD)(
        y1.reshape(2 * NPAD, 128), src1, dst1
    )

    y2 = _tc2(acc1, y1, degT, W2, b1.reshape(1, 256))

    nch2 = E // (NC * NS * CH)
    src2 = src.reshape(NC, NS, nch2 // GRP, GRP, CH)
    dst2 = dst.reshape(NC, NS, nch2 // GRP, GRP, CH)
    acc2 = _make_prop(nch2, NPAD)(y2, src2, dst2)

    out = _tc3(acc2, y2, degT, b2.reshape(1, 128), Wc, bc.reshape(1, 10))
    return out[:N]


# speedup vs baseline: 23.3909x; 1.1225x over previous
import jax
import jax.numpy as jnp
from jax import lax
from jax.experimental import pallas as pl
from jax.experimental.pallas import tpu as pltpu
from jax.experimental.pallas import tpu_sc as plsc

N = 10000
NPAD = 10240
NB = 1024
E = 320000
NC = 2
NS = 16
CH = 125
GRP = 20
ZB = 32
RPT = NPAD // NS
HR = NPAD // 128
HRT = HR // NS
F32 = jnp.float32


_MESH = dict(core_axis_name="c", subcore_axis_name="s")
_SC_PARAMS = pltpu.CompilerParams(needs_layout_passes=False)
_EPT_DEG = E // (NC * NS)


def _deg_body(dst_hbm, out_hbm, idx_v, hist_v, iv, acc_sh):
    c = lax.axis_index("c")
    s = lax.axis_index("s")
    wid = c * NS + s
    pltpu.sync_copy(dst_hbm.at[wid], idx_v)

    def zrow(r, carry):
        def zcol(k, carry2):
            hist_v[r, pl.ds(k * 16, 16)] = jnp.zeros((16,), F32)
            return carry2

        return lax.fori_loop(0, 8, zcol, carry)

    lax.fori_loop(0, HR, zrow, 0)

    def fill(j, carry):
        iv[0, pl.ds(j * 16, 16)] = lax.iota(jnp.int32, 16) + j * 16
        return carry

    lax.fori_loop(0, HR // 16, fill, 0)
    pltpu.sync_copy(hist_v.at[pl.ds(0, HRT)], acc_sh.at[pl.ds(s * HRT, HRT)])
    plsc.subcore_barrier()

    ones = jnp.ones((16,), F32)

    def acc(i, carry):
        idx = idx_v[0, pl.ds(i * 16, 16)]
        rows = lax.shift_right_logical(idx, 7)
        lanes = lax.bitwise_and(idx, 127)
        plsc.addupdate_scatter(hist_v, [rows, lanes], ones)
        return carry

    lax.fori_loop(0, _EPT_DEG // 16, acc, 0)

    pltpu.sync_copy(hist_v, acc_sh.at[iv.at[0]], add=True)
    plsc.subcore_barrier()
    pltpu.sync_copy(acc_sh.at[pl.ds(s * HRT, HRT)], hist_v.at[pl.ds(0, HRT)])
    pltpu.sync_copy(hist_v.at[pl.ds(0, HRT)], out_hbm.at[c, s])


def _deg(dst32):
    k = pl.kernel(
        _deg_body,
        out_type=jax.ShapeDtypeStruct((NC, NS, HRT, 128), F32),
        mesh=plsc.VectorSubcoreMesh(**_MESH),
        compiler_params=_SC_PARAMS,
        scratch_types=[
            pltpu.VMEM((1, _EPT_DEG), jnp.int32),
            pltpu.VMEM((HR, 128), F32),
            pltpu.VMEM((1, HR), jnp.int32),
            pltpu.VMEM_SHARED((HR, 128), F32),
        ],
    )
    return k(dst32.reshape(NC * NS, 1, _EPT_DEG))


def _make_prop(n_chunks, feat_split):
    def body(table, src_hbm, dst_hbm, out_hbm, src_v, dst_v, g0, g1, zb,
             acc_sh, sem0, sem1):
        c = lax.axis_index("c")
        s = lax.axis_index("s")
        tc = c if feat_split else 0

        def zrow(r, carry):
            def zcol(k, carry2):
                zb[r, pl.ds(k * 16, 16)] = jnp.zeros((16,), F32)
                return carry2

            return lax.fori_loop(0, 8, zcol, carry)

        lax.fori_loop(0, ZB, zrow, 0)

        def zcp(j, carry):
            pltpu.sync_copy(zb, acc_sh.at[pl.ds(s * RPT + j * ZB, ZB)])
            return carry

        lax.fori_loop(0, RPT // ZB, zcp, 0)
        plsc.subcore_barrier()

        def do_pair(j0, prefetch):
            j1 = j0 + 1
            pltpu.make_async_copy(table.at[tc].at[src_v.at[j0]], g0, sem0).wait()
            pltpu.async_copy(table.at[tc].at[src_v.at[j1]], g1, sem1)
            pltpu.sync_copy(g0, acc_sh.at[dst_v.at[j0]], add=True)
            pltpu.make_async_copy(table.at[tc].at[src_v.at[j1]], g1, sem1).wait()
            if prefetch:
                pltpu.async_copy(table.at[tc].at[src_v.at[j0 + 2]], g0, sem0)
            pltpu.sync_copy(g1, acc_sh.at[dst_v.at[j1]], add=True)

        def group(g, carry):
            if feat_split:
                pltpu.sync_copy(src_hbm.at[s, g], src_v)
                pltpu.sync_copy(dst_hbm.at[s, g], dst_v)
            else:
                pltpu.sync_copy(src_hbm.at[c, s, g], src_v)
                pltpu.sync_copy(dst_hbm.at[c, s, g], dst_v)
            pltpu.async_copy(table.at[tc].at[src_v.at[0]], g0, sem0)

            def pair_loop(p, carry2):
                do_pair(p * 2, True)
                return carry2

            lax.fori_loop(0, GRP // 2 - 1, pair_loop, carry)
            do_pair(GRP - 2, False)
            return carry

        lax.fori_loop(0, n_chunks // GRP, group, 0)
        plsc.subcore_barrier()

        def wb(j, carry):
            base = s * RPT + j * ZB
            pltpu.sync_copy(acc_sh.at[pl.ds(base, ZB)], zb)
            pltpu.sync_copy(zb, out_hbm.at[c, pl.ds(base, ZB)])
            return carry

        lax.fori_loop(0, RPT // ZB, wb, 0)

    return pl.kernel(
        body,
        out_type=jax.ShapeDtypeStruct((NC, NPAD, 128), F32),
        mesh=plsc.VectorSubcoreMesh(**_MESH),
        compiler_params=_SC_PARAMS,
        scratch_types=[
            pltpu.VMEM((GRP, CH), jnp.int32),
            pltpu.VMEM((GRP, CH), jnp.int32),
            pltpu.VMEM((CH, 128), F32),
            pltpu.VMEM((CH, 128), F32),
            pltpu.VMEM((ZB, 128), F32),
            pltpu.VMEM_SHARED((NPAD, 128), F32),
            pltpu.SemaphoreType.DMA,
            pltpu.SemaphoreType.DMA,
        ],
    )


def _dinv_of(dT_blk):
    return lax.rsqrt(jnp.sum(dT_blk, axis=1, keepdims=True) + 1.0)


def _tc1_body(x_ref, w_ref, dT_ref, y_ref):
    dinv = _dinv_of(dT_ref[...])
    xw = jnp.dot(x_ref[...], w_ref[...], preferred_element_type=F32)
    y = xw * dinv
    y_ref[0] = y[:, :128]
    y_ref[1] = y[:, 128:]


def _tc1(x, W1, degT, interpret=False):
    return pl.pallas_call(
        _tc1_body,
        grid=(NPAD // NB,),
        in_specs=[
            pl.BlockSpec((NB, 128), lambda i: (i, 0)),
            pl.BlockSpec((128, 256), lambda i: (0, 0)),
            pl.BlockSpec((NB, NC), lambda i: (i, 0)),
        ],
        out_specs=pl.BlockSpec((2, NB, 128), lambda i: (0, i, 0)),
        out_shape=jax.ShapeDtypeStruct((2, NPAD, 128), F32),
        interpret=interpret,
    )(x, W1, degT)


def _tc2_body(a_ref, y_ref, dT_ref, w2_ref, b1_ref, o_ref):
    dinv = _dinv_of(dT_ref[...])
    h = jnp.concatenate([a_ref[0] + y_ref[0], a_ref[1] + y_ref[1]], axis=1)
    h = jnp.maximum(h * dinv + b1_ref[...], 0.0)
    o_ref[...] = jnp.dot(h, w2_ref[...], preferred_element_type=F32) * dinv


def _tc2(acc1, y1, degT, W2, b1r, interpret=False):
    return pl.pallas_call(
        _tc2_body,
        grid=(NPAD // NB,),
        in_specs=[
            pl.BlockSpec((2, NB, 128), lambda i: (0, i, 0)),
            pl.BlockSpec((2, NB, 128), lambda i: (0, i, 0)),
            pl.BlockSpec((NB, NC), lambda i: (i, 0)),
            pl.BlockSpec((256, 128), lambda i: (0, 0)),
            pl.BlockSpec((1, 256), lambda i: (0, 0)),
        ],
        out_specs=pl.BlockSpec((NB, 128), lambda i: (i, 0)),
        out_shape=jax.ShapeDtypeStruct((NPAD, 128), F32),
        interpret=interpret,
    )(acc1, y1, degT, W2, b1r)


def _tc3_body(a_ref, y_ref, dT_ref, b2_ref, wc_ref, bc_ref, o_ref):
    dinv = _dinv_of(dT_ref[...])
    h = jnp.maximum((a_ref[0] + a_ref[1] + y_ref[...]) * dinv + b2_ref[...], 0.0)
    logits = jnp.dot(h, wc_ref[...], preferred_element_type=F32) + bc_ref[...]
    m = jnp.max(logits, axis=1, keepdims=True)
    lse = jnp.log(jnp.sum(jnp.exp(logits - m), axis=1, keepdims=True))
    o_ref[...] = logits - m - lse


def _tc3(acc2, y2, degT, b2r, Wc, bcr, interpret=False):
    return pl.pallas_call(
        _tc3_body,
        grid=(NPAD // NB,),
        in_specs=[
            pl.BlockSpec((2, NB, 128), lambda i: (0, i, 0)),
            pl.BlockSpec((NB, 128), lambda i: (i, 0)),
            pl.BlockSpec((NB, NC), lambda i: (i, 0)),
            pl.BlockSpec((1, 128), lambda i: (0, 0)),
            pl.BlockSpec((128, 10), lambda i: (0, 0)),
            pl.BlockSpec((1, 10), lambda i: (0, 0)),
        ],
        out_specs=pl.BlockSpec((NB, 10), lambda i: (i, 0)),
        out_shape=jax.ShapeDtypeStruct((N, 10), F32),
        interpret=interpret,
    )(acc2, y2, degT, b2r, Wc, bcr)


def kernel(x, edge_index, W1, b1, W2, b2, Wc, bc):
    ei = edge_index.astype(jnp.int32)
    src = ei[0]
    dst = ei[1]

    degp = _deg(dst).reshape(NC, NPAD)
    degT = degp.T

    y1 = _tc1(x, W1, degT)

    nch1 = E // (NS * CH)
    src1 = src.reshape(NS, nch1 // GRP, GRP, CH)
    dst1 = dst.reshape(NS, nch1 // GRP, GRP, CH)
    acc1 = _make_prop(nch1, True)(y1, src1, dst1)

    y2 = _tc2(acc1, y1, degT, W2, b1.reshape(1, 256))

    nch2 = E // (NC * NS * CH)
    src2 = src.reshape(NC, NS, nch2 // GRP, GRP, CH)
    dst2 = dst.reshape(NC, NS, nch2 // GRP, GRP, CH)
    acc2 = _make_prop(nch2, False)(y2.reshape(1, NPAD, 128), src2, dst2)

    return _tc3(acc2, y2, degT, b2.reshape(1, 128), Wc, bc.reshape(1, 10))


# speedup vs baseline: 23.5419x; 1.0065x over previous
import jax
import jax.numpy as jnp
from jax import lax
from jax.experimental import pallas as pl
from jax.experimental.pallas import tpu as pltpu
from jax.experimental.pallas import tpu_sc as plsc

N = 10000
NPAD = 10240
NB = 1024
E = 320000
NC = 2
NS = 16
CH = 125
GRP = 20
ZB = 32
RPT = NPAD // NS
HR = NPAD // 128
HRT = HR // NS
F32 = jnp.float32


_MESH = dict(core_axis_name="c", subcore_axis_name="s")
_SC_PARAMS = pltpu.CompilerParams(needs_layout_passes=False)
_EPT_DEG = E // (NC * NS)


def _deg_body(dst_hbm, out_hbm, idx_v, hist_v, iv, acc_sh):
    c = lax.axis_index("c")
    s = lax.axis_index("s")
    wid = c * NS + s
    pltpu.sync_copy(dst_hbm.at[wid], idx_v)

    def zrow(r, carry):
        def zcol(k, carry2):
            hist_v[r, pl.ds(k * 16, 16)] = jnp.zeros((16,), F32)
            return carry2

        return lax.fori_loop(0, 8, zcol, carry)

    lax.fori_loop(0, HR, zrow, 0)

    def fill(j, carry):
        iv[0, pl.ds(j * 16, 16)] = lax.iota(jnp.int32, 16) + j * 16
        return carry

    lax.fori_loop(0, HR // 16, fill, 0)
    pltpu.sync_copy(hist_v.at[pl.ds(0, HRT)], acc_sh.at[pl.ds(s * HRT, HRT)])
    plsc.subcore_barrier()

    ones = jnp.ones((16,), F32)

    def acc(i, carry):
        idx = idx_v[0, pl.ds(i * 16, 16)]
        rows = lax.shift_right_logical(idx, 7)
        lanes = lax.bitwise_and(idx, 127)
        plsc.addupdate_scatter(hist_v, [rows, lanes], ones)
        return carry

    lax.fori_loop(0, _EPT_DEG // 16, acc, 0)

    pltpu.sync_copy(hist_v, acc_sh.at[iv.at[0]], add=True)
    plsc.subcore_barrier()
    pltpu.sync_copy(acc_sh.at[pl.ds(s * HRT, HRT)], hist_v.at[pl.ds(0, HRT)])
    pltpu.sync_copy(hist_v.at[pl.ds(0, HRT)], out_hbm.at[c, s])


def _deg(dst32):
    k = pl.kernel(
        _deg_body,
        out_type=jax.ShapeDtypeStruct((NC, NS, HRT, 128), F32),
        mesh=plsc.VectorSubcoreMesh(**_MESH),
        compiler_params=_SC_PARAMS,
        scratch_types=[
            pltpu.VMEM((1, _EPT_DEG), jnp.int32),
            pltpu.VMEM((HR, 128), F32),
            pltpu.VMEM((1, HR), jnp.int32),
            pltpu.VMEM_SHARED((HR, 128), F32),
        ],
    )
    return k(dst32.reshape(NC * NS, 1, _EPT_DEG))


def _make_prop(n_chunks, feat_split):
    def body(table, src_hbm, dst_hbm, out_hbm, src_v, dst_v, g0, g1, zb,
             acc_sh, sem0, sem1):
        c = lax.axis_index("c")
        s = lax.axis_index("s")
        tc = c if feat_split else 0

        def zrow(r, carry):
            def zcol(k, carry2):
                zb[r, pl.ds(k * 16, 16)] = jnp.zeros((16,), F32)
                return carry2

            return lax.fori_loop(0, 8, zcol, carry)

        lax.fori_loop(0, ZB, zrow, 0)

        def zcp(j, carry):
            pltpu.sync_copy(zb, acc_sh.at[pl.ds(s * RPT + j * ZB, ZB)])
            return carry

        lax.fori_loop(0, RPT // ZB, zcp, 0)
        plsc.subcore_barrier()

        def do_pair(j0, prefetch):
            j1 = j0 + 1
            pltpu.make_async_copy(table.at[tc].at[src_v.at[j0]], g0, sem0).wait()
            pltpu.async_copy(table.at[tc].at[src_v.at[j1]], g1, sem1)
            pltpu.sync_copy(g0, acc_sh.at[dst_v.at[j0]], add=True)
            pltpu.make_async_copy(table.at[tc].at[src_v.at[j1]], g1, sem1).wait()
            if prefetch:
                pltpu.async_copy(table.at[tc].at[src_v.at[j0 + 2]], g0, sem0)
            pltpu.sync_copy(g1, acc_sh.at[dst_v.at[j1]], add=True)

        def group(g, carry):
            if feat_split:
                pltpu.sync_copy(src_hbm.at[s, g], src_v)
                pltpu.sync_copy(dst_hbm.at[s, g], dst_v)
            else:
                pltpu.sync_copy(src_hbm.at[c, s, g], src_v)
                pltpu.sync_copy(dst_hbm.at[c, s, g], dst_v)
            pltpu.async_copy(table.at[tc].at[src_v.at[0]], g0, sem0)

            def pair_loop(p, carry2):
                do_pair(p * 2, True)
                return carry2

            lax.fori_loop(0, GRP // 2 - 1, pair_loop, carry)
            do_pair(GRP - 2, False)
            return carry

        lax.fori_loop(0, n_chunks // GRP, group, 0)
        plsc.subcore_barrier()

        pltpu.sync_copy(acc_sh.at[pl.ds(s * RPT, RPT)],
                        out_hbm.at[c, pl.ds(s * RPT, RPT)])

    return pl.kernel(
        body,
        out_type=jax.ShapeDtypeStruct((NC, NPAD, 128), F32),
        mesh=plsc.VectorSubcoreMesh(**_MESH),
        compiler_params=_SC_PARAMS,
        scratch_types=[
            pltpu.VMEM((GRP, CH), jnp.int32),
            pltpu.VMEM((GRP, CH), jnp.int32),
            pltpu.VMEM((CH, 128), F32),
            pltpu.VMEM((CH, 128), F32),
            pltpu.VMEM((ZB, 128), F32),
            pltpu.VMEM_SHARED((NPAD, 128), F32),
            pltpu.SemaphoreType.DMA,
            pltpu.SemaphoreType.DMA,
        ],
    )


def _dinv_of(dT_blk):
    return lax.rsqrt(jnp.sum(dT_blk, axis=1, keepdims=True) + 1.0)


def _tc1_body(x_ref, w_ref, dT_ref, y_ref):
    dinv = _dinv_of(dT_ref[...])
    xw = jnp.dot(x_ref[...], w_ref[...], preferred_element_type=F32)
    y = xw * dinv
    y_ref[0] = y[:, :128]
    y_ref[1] = y[:, 128:]


def _tc1(x, W1, degT, interpret=False):
    return pl.pallas_call(
        _tc1_body,
        grid=(NPAD // NB,),
        in_specs=[
            pl.BlockSpec((NB, 128), lambda i: (i, 0)),
            pl.BlockSpec((128, 256), lambda i: (0, 0)),
            pl.BlockSpec((NB, NC), lambda i: (i, 0)),
        ],
        out_specs=pl.BlockSpec((2, NB, 128), lambda i: (0, i, 0)),
        out_shape=jax.ShapeDtypeStruct((2, NPAD, 128), F32),
        interpret=interpret,
    )(x, W1, degT)


def _tc2_body(a_ref, y_ref, dT_ref, w2_ref, b1_ref, o_ref):
    dinv = _dinv_of(dT_ref[...])
    h = jnp.concatenate([a_ref[0] + y_ref[0], a_ref[1] + y_ref[1]], axis=1)
    h = jnp.maximum(h * dinv + b1_ref[...], 0.0)
    o_ref[...] = jnp.dot(h, w2_ref[...], preferred_element_type=F32) * dinv


def _tc2(acc1, y1, degT, W2, b1r, interpret=False):
    return pl.pallas_call(
        _tc2_body,
        grid=(NPAD // NB,),
        in_specs=[
            pl.BlockSpec((2, NB, 128), lambda i: (0, i, 0)),
            pl.BlockSpec((2, NB, 128), lambda i: (0, i, 0)),
            pl.BlockSpec((NB, NC), lambda i: (i, 0)),
            pl.BlockSpec((256, 128), lambda i: (0, 0)),
            pl.BlockSpec((1, 256), lambda i: (0, 0)),
        ],
        out_specs=pl.BlockSpec((NB, 128), lambda i: (i, 0)),
        out_shape=jax.ShapeDtypeStruct((NPAD, 128), F32),
        interpret=interpret,
    )(acc1, y1, degT, W2, b1r)


def _tc3_body(a_ref, y_ref, dT_ref, b2_ref, wc_ref, bc_ref, o_ref):
    dinv = _dinv_of(dT_ref[...])
    h = jnp.maximum((a_ref[0] + a_ref[1] + y_ref[...]) * dinv + b2_ref[...], 0.0)
    logits = jnp.dot(h, wc_ref[...], preferred_element_type=F32) + bc_ref[...]
    m = jnp.max(logits, axis=1, keepdims=True)
    lse = jnp.log(jnp.sum(jnp.exp(logits - m), axis=1, keepdims=True))
    o_ref[...] = logits - m - lse


def _tc3(acc2, y2, degT, b2r, Wc, bcr, interpret=False):
    return pl.pallas_call(
        _tc3_body,
        grid=(NPAD // NB,),
        in_specs=[
            pl.BlockSpec((2, NB, 128), lambda i: (0, i, 0)),
            pl.BlockSpec((NB, 128), lambda i: (i, 0)),
            pl.BlockSpec((NB, NC), lambda i: (i, 0)),
            pl.BlockSpec((1, 128), lambda i: (0, 0)),
            pl.BlockSpec((128, 10), lambda i: (0, 0)),
            pl.BlockSpec((1, 10), lambda i: (0, 0)),
        ],
        out_specs=pl.BlockSpec((NB, 10), lambda i: (i, 0)),
        out_shape=jax.ShapeDtypeStruct((N, 10), F32),
        interpret=interpret,
    )(acc2, y2, degT, b2r, Wc, bcr)


def kernel(x, edge_index, W1, b1, W2, b2, Wc, bc):
    ei = edge_index.astype(jnp.int32)
    src = ei[0]
    dst = ei[1]

    degp = _deg(dst).reshape(NC, NPAD)
    degT = degp.T

    y1 = _tc1(x, W1, degT)

    nch1 = E // (NS * CH)
    src1 = src.reshape(NS, nch1 // GRP, GRP, CH)
    dst1 = dst.reshape(NS, nch1 // GRP, GRP, CH)
    acc1 = _make_prop(nch1, True)(y1, src1, dst1)

    y2 = _tc2(acc1, y1, degT, W2, b1.reshape(1, 256))

    nch2 = E // (NC * NS * CH)
    src2 = src.reshape(NC, NS, nch2 // GRP, GRP, CH)
    dst2 = dst.reshape(NC, NS, nch2 // GRP, GRP, CH)
    acc2 = _make_prop(nch2, False)(y2.reshape(1, NPAD, 128), src2, dst2)

    return _tc3(acc2, y2, degT, b2.reshape(1, 128), Wc, bc.reshape(1, 10))


# speedup vs baseline: 24.8819x; 1.0569x over previous
import jax
import jax.numpy as jnp
from jax import lax
from jax.experimental import pallas as pl
from jax.experimental.pallas import tpu as pltpu
from jax.experimental.pallas import tpu_sc as plsc

N = 10000
NPAD = 10240
NB = 1024
E = 320000
NC = 2
NS = 16
CH = 125
GRP = 10
ZB = 32
RPT = NPAD // NS
HR = NPAD // 128
HRT = HR // NS
F32 = jnp.float32


_MESH = dict(core_axis_name="c", subcore_axis_name="s")
_SC_PARAMS = pltpu.CompilerParams(needs_layout_passes=False)
_EPT_DEG = E // (NC * NS)


def _deg_body(dst_hbm, out_hbm, idx_v, hist_v, iv, acc_sh):
    c = lax.axis_index("c")
    s = lax.axis_index("s")
    wid = c * NS + s
    pltpu.sync_copy(dst_hbm.at[wid], idx_v)

    def zrow(r, carry):
        def zcol(k, carry2):
            hist_v[r, pl.ds(k * 16, 16)] = jnp.zeros((16,), F32)
            return carry2

        return lax.fori_loop(0, 8, zcol, carry)

    lax.fori_loop(0, HR, zrow, 0)

    def fill(j, carry):
        iv[0, pl.ds(j * 16, 16)] = lax.iota(jnp.int32, 16) + j * 16
        return carry

    lax.fori_loop(0, HR // 16, fill, 0)
    pltpu.sync_copy(hist_v.at[pl.ds(0, HRT)], acc_sh.at[pl.ds(s * HRT, HRT)])
    plsc.subcore_barrier()

    ones = jnp.ones((16,), F32)

    def acc(i, carry):
        idx = idx_v[0, pl.ds(i * 16, 16)]
        rows = lax.shift_right_logical(idx, 7)
        lanes = lax.bitwise_and(idx, 127)
        plsc.addupdate_scatter(hist_v, [rows, lanes], ones)
        return carry

    lax.fori_loop(0, _EPT_DEG // 16, acc, 0)

    pltpu.sync_copy(hist_v, acc_sh.at[iv.at[0]], add=True)
    plsc.subcore_barrier()
    pltpu.sync_copy(acc_sh.at[pl.ds(s * HRT, HRT)], hist_v.at[pl.ds(0, HRT)])
    pltpu.sync_copy(hist_v.at[pl.ds(0, HRT)], out_hbm.at[c, s])


def _deg(dst32):
    k = pl.kernel(
        _deg_body,
        out_type=jax.ShapeDtypeStruct((NC, NS, HRT, 128), F32),
        mesh=plsc.VectorSubcoreMesh(**_MESH),
        compiler_params=_SC_PARAMS,
        scratch_types=[
            pltpu.VMEM((1, _EPT_DEG), jnp.int32),
            pltpu.VMEM((HR, 128), F32),
            pltpu.VMEM((1, HR), jnp.int32),
            pltpu.VMEM_SHARED((HR, 128), F32),
        ],
    )
    return k(dst32.reshape(NC * NS, 1, _EPT_DEG))


def _make_prop(n_chunks, feat_split):
    NG = n_chunks // GRP

    def body(table, src_hbm, dst_hbm, out_hbm, sva, dva, svb, dvb, g0, g1,
             zb, acc_sh, sem0, sem1, isem):
        c = lax.axis_index("c")
        s = lax.axis_index("s")
        tc = c if feat_split else 0

        def load_idx(g, sv, dv):
            if feat_split:
                pltpu.async_copy(src_hbm.at[s, g], sv, isem)
                pltpu.async_copy(dst_hbm.at[s, g], dv, isem)
            else:
                pltpu.async_copy(src_hbm.at[c, s, g], sv, isem)
                pltpu.async_copy(dst_hbm.at[c, s, g], dv, isem)

        def wait_idx(g, sv, dv):
            if feat_split:
                pltpu.make_async_copy(src_hbm.at[s, g], sv, isem).wait()
                pltpu.make_async_copy(dst_hbm.at[s, g], dv, isem).wait()
            else:
                pltpu.make_async_copy(src_hbm.at[c, s, g], sv, isem).wait()
                pltpu.make_async_copy(dst_hbm.at[c, s, g], dv, isem).wait()

        def zrow(r, carry):
            def zcol(k, carry2):
                zb[r, pl.ds(k * 16, 16)] = jnp.zeros((16,), F32)
                return carry2

            return lax.fori_loop(0, 8, zcol, carry)

        lax.fori_loop(0, ZB, zrow, 0)
        load_idx(0, sva, dva)

        def zcp(j, carry):
            pltpu.async_copy(zb, acc_sh.at[pl.ds(s * RPT + j * ZB, ZB)], sem0)
            return carry

        lax.fori_loop(0, RPT // ZB, zcp, 0)

        def zwt(j, carry):
            pltpu.make_async_copy(
                zb, acc_sh.at[pl.ds(s * RPT + j * ZB, ZB)], sem0).wait()
            return carry

        lax.fori_loop(0, RPT // ZB, zwt, 0)
        wait_idx(0, sva, dva)
        pltpu.async_copy(table.at[tc].at[sva.at[0]], g0, sem0)
        plsc.subcore_barrier()

        def do_pair(sv, dv, j0, pf):
            j1 = j0 + 1
            pltpu.make_async_copy(table.at[tc].at[sv.at[j0]], g0, sem0).wait()
            pltpu.async_copy(table.at[tc].at[sv.at[j1]], g1, sem1)
            pltpu.sync_copy(g0, acc_sh.at[dv.at[j0]], add=True)
            pltpu.make_async_copy(table.at[tc].at[sv.at[j1]], g1, sem1).wait()
            if pf is not None:
                pltpu.async_copy(table.at[tc].at[pf[0].at[pf[1]]], g0, sem0)
            pltpu.sync_copy(g1, acc_sh.at[dv.at[j1]], add=True)

        def mid_pairs(sv, dv):
            def pp(p, carry2):
                do_pair(sv, dv, p * 2, (sv, p * 2 + 2))
                return carry2

            lax.fori_loop(0, GRP // 2 - 1, pp, 0)

        def two_groups(k, carry):
            gb = 2 * k + 1
            gn = jnp.minimum(gb + 1, NG - 1)
            load_idx(gb, svb, dvb)
            mid_pairs(sva, dva)
            wait_idx(gb, svb, dvb)
            do_pair(sva, dva, GRP - 2, (svb, 0))
            load_idx(gn, sva, dva)
            mid_pairs(svb, dvb)
            wait_idx(gn, sva, dva)
            do_pair(svb, dvb, GRP - 2, (sva, 0))
            return carry

        lax.fori_loop(0, NG // 2 - 1, two_groups, 0)
        load_idx(NG - 1, svb, dvb)
        mid_pairs(sva, dva)
        wait_idx(NG - 1, svb, dvb)
        do_pair(sva, dva, GRP - 2, (svb, 0))
        mid_pairs(svb, dvb)
        do_pair(svb, dvb, GRP - 2, None)
        plsc.subcore_barrier()

        pltpu.sync_copy(acc_sh.at[pl.ds(s * RPT, RPT)],
                        out_hbm.at[c, pl.ds(s * RPT, RPT)])

    return pl.kernel(
        body,
        out_type=jax.ShapeDtypeStruct((NC, NPAD, 128), F32),
        mesh=plsc.VectorSubcoreMesh(**_MESH),
        compiler_params=_SC_PARAMS,
        scratch_types=[
            pltpu.VMEM((GRP, CH), jnp.int32),
            pltpu.VMEM((GRP, CH), jnp.int32),
            pltpu.VMEM((GRP, CH), jnp.int32),
            pltpu.VMEM((GRP, CH), jnp.int32),
            pltpu.VMEM((CH, 128), F32),
            pltpu.VMEM((CH, 128), F32),
            pltpu.VMEM((ZB, 128), F32),
            pltpu.VMEM_SHARED((NPAD, 128), F32),
            pltpu.SemaphoreType.DMA,
            pltpu.SemaphoreType.DMA,
            pltpu.SemaphoreType.DMA,
        ],
    )


def _dinv_of(dT_blk):
    return lax.rsqrt(jnp.sum(dT_blk, axis=1, keepdims=True) + 1.0)


def _tc1_body(x_ref, w_ref, dT_ref, y_ref):
    dinv = _dinv_of(dT_ref[...])
    xw = jnp.dot(x_ref[...], w_ref[...], preferred_element_type=F32)
    y = xw * dinv
    y_ref[0] = y[:, :128]
    y_ref[1] = y[:, 128:]


def _tc1(x, W1, degT, interpret=False):
    return pl.pallas_call(
        _tc1_body,
        grid=(NPAD // NB,),
        in_specs=[
            pl.BlockSpec((NB, 128), lambda i: (i, 0)),
            pl.BlockSpec((128, 256), lambda i: (0, 0)),
            pl.BlockSpec((NB, NC), lambda i: (i, 0)),
        ],
        out_specs=pl.BlockSpec((2, NB, 128), lambda i: (0, i, 0)),
        out_shape=jax.ShapeDtypeStruct((2, NPAD, 128), F32),
        interpret=interpret,
    )(x, W1, degT)


def _tc2_body(a_ref, y_ref, dT_ref, w2_ref, b1_ref, o_ref):
    dinv = _dinv_of(dT_ref[...])
    h = jnp.concatenate([a_ref[0] + y_ref[0], a_ref[1] + y_ref[1]], axis=1)
    h = jnp.maximum(h * dinv + b1_ref[...], 0.0)
    o_ref[...] = jnp.dot(h, w2_ref[...], preferred_element_type=F32) * dinv


def _tc2(acc1, y1, degT, W2, b1r, interpret=False):
    return pl.pallas_call(
        _tc2_body,
        grid=(NPAD // NB,),
        in_specs=[
            pl.BlockSpec((2, NB, 128), lambda i: (0, i, 0)),
            pl.BlockSpec((2, NB, 128), lambda i: (0, i, 0)),
            pl.BlockSpec((NB, NC), lambda i: (i, 0)),
            pl.BlockSpec((256, 128), lambda i: (0, 0)),
            pl.BlockSpec((1, 256), lambda i: (0, 0)),
        ],
        out_specs=pl.BlockSpec((NB, 128), lambda i: (i, 0)),
        out_shape=jax.ShapeDtypeStruct((NPAD, 128), F32),
        interpret=interpret,
    )(acc1, y1, degT, W2, b1r)


def _tc3_body(a_ref, y_ref, dT_ref, b2_ref, wc_ref, bc_ref, o_ref):
    dinv = _dinv_of(dT_ref[...])
    h = jnp.maximum((a_ref[0] + a_ref[1] + y_ref[...]) * dinv + b2_ref[...], 0.0)
    logits = jnp.dot(h, wc_ref[...], preferred_element_type=F32) + bc_ref[...]
    m = jnp.max(logits, axis=1, keepdims=True)
    lse = jnp.log(jnp.sum(jnp.exp(logits - m), axis=1, keepdims=True))
    o_ref[...] = logits - m - lse


def _tc3(acc2, y2, degT, b2r, Wc, bcr, interpret=False):
    return pl.pallas_call(
        _tc3_body,
        grid=(NPAD // NB,),
        in_specs=[
            pl.BlockSpec((2, NB, 128), lambda i: (0, i, 0)),
            pl.BlockSpec((NB, 128), lambda i: (i, 0)),
            pl.BlockSpec((NB, NC), lambda i: (i, 0)),
            pl.BlockSpec((1, 128), lambda i: (0, 0)),
            pl.BlockSpec((128, 10), lambda i: (0, 0)),
            pl.BlockSpec((1, 10), lambda i: (0, 0)),
        ],
        out_specs=pl.BlockSpec((NB, 10), lambda i: (i, 0)),
        out_shape=jax.ShapeDtypeStruct((N, 10), F32),
        interpret=interpret,
    )(acc2, y2, degT, b2r, Wc, bcr)


def kernel(x, edge_index, W1, b1, W2, b2, Wc, bc):
    ei = edge_index.astype(jnp.int32)
    src = ei[0]
    dst = ei[1]

    degp = _deg(dst).reshape(NC, NPAD)
    degT = degp.T

    y1 = _tc1(x, W1, degT)

    nch1 = E // (NS * CH)
    src1 = src.reshape(NS, nch1 // GRP, GRP, CH)
    dst1 = dst.reshape(NS, nch1 // GRP, GRP, CH)
    acc1 = _make_prop(nch1, True)(y1, src1, dst1)

    y2 = _tc2(acc1, y1, degT, W2, b1.reshape(1, 256))

    nch2 = E // (NC * NS * CH)
    src2 = src.reshape(NC, NS, nch2 // GRP, GRP, CH)
    dst2 = dst.reshape(NC, NS, nch2 // GRP, GRP, CH)
    acc2 = _make_prop(nch2, False)(y2.reshape(1, NPAD, 128), src2, dst2)

    return _tc3(acc2, y2, degT, b2.reshape(1, 128), Wc, bc.reshape(1, 10))


# speedup vs baseline: 29.0633x; 1.1680x over previous
import jax
import jax.numpy as jnp
from jax import lax
from jax.experimental import pallas as pl
from jax.experimental.pallas import tpu as pltpu
from jax.experimental.pallas import tpu_sc as plsc

N = 10000
NPAD = 10240
NB = 1024
E = 320000
NC = 2
NS = 16
CH = 125
GRP = 10
ZB = 32
RPT = NPAD // NS
HR = NPAD // 128
HRT = HR // NS
F32 = jnp.float32


_MESH = dict(core_axis_name="c", subcore_axis_name="s")
_SC_PARAMS = pltpu.CompilerParams(needs_layout_passes=False)
_EPT_DEG = E // (NC * NS)


def _deg_body(dst_hbm, out_hbm, idx_v, hist_v, iv, acc_sh):
    c = lax.axis_index("c")
    s = lax.axis_index("s")
    wid = c * NS + s
    pltpu.sync_copy(dst_hbm.at[wid], idx_v)

    def zrow(r, carry):
        def zcol(k, carry2):
            hist_v[r, pl.ds(k * 16, 16)] = jnp.zeros((16,), F32)
            return carry2

        return lax.fori_loop(0, 8, zcol, carry)

    lax.fori_loop(0, HR, zrow, 0)

    def fill(j, carry):
        iv[0, pl.ds(j * 16, 16)] = lax.iota(jnp.int32, 16) + j * 16
        return carry

    lax.fori_loop(0, HR // 16, fill, 0)
    pltpu.sync_copy(hist_v.at[pl.ds(0, HRT)], acc_sh.at[pl.ds(s * HRT, HRT)])
    plsc.subcore_barrier()

    ones = jnp.ones((16,), F32)

    def acc(i, carry):
        idx = idx_v[0, pl.ds(i * 16, 16)]
        rows = lax.shift_right_logical(idx, 7)
        lanes = lax.bitwise_and(idx, 127)
        plsc.addupdate_scatter(hist_v, [rows, lanes], ones)
        return carry

    lax.fori_loop(0, _EPT_DEG // 16, acc, 0)

    pltpu.sync_copy(hist_v, acc_sh.at[iv.at[0]], add=True)
    plsc.subcore_barrier()
    pltpu.sync_copy(acc_sh.at[pl.ds(s * HRT, HRT)], hist_v.at[pl.ds(0, HRT)])
    pltpu.sync_copy(hist_v.at[pl.ds(0, HRT)], out_hbm.at[c, s])


def _deg(dst32):
    k = pl.kernel(
        _deg_body,
        out_type=jax.ShapeDtypeStruct((NC, NS, HRT, 128), F32),
        mesh=plsc.VectorSubcoreMesh(**_MESH),
        compiler_params=_SC_PARAMS,
        scratch_types=[
            pltpu.VMEM((1, _EPT_DEG), jnp.int32),
            pltpu.VMEM((HR, 128), F32),
            pltpu.VMEM((1, HR), jnp.int32),
            pltpu.VMEM_SHARED((HR, 128), F32),
        ],
    )
    return k(dst32.reshape(NC * NS, 1, _EPT_DEG))


def _make_prop(n_chunks, feat_split):
    NG = n_chunks // GRP

    def body(table, src_hbm, dst_hbm, out_hbm, sva, dva, svb, dvb, g0, g1,
             zb, acc_sh, sem0, sem1, isem):
        c = lax.axis_index("c")
        s = lax.axis_index("s")
        tc = c if feat_split else 0

        def load_idx(g, sv, dv):
            if feat_split:
                pltpu.async_copy(src_hbm.at[s, g], sv, isem)
                pltpu.async_copy(dst_hbm.at[s, g], dv, isem)
            else:
                pltpu.async_copy(src_hbm.at[c, s, g], sv, isem)
                pltpu.async_copy(dst_hbm.at[c, s, g], dv, isem)

        def wait_idx(g, sv, dv):
            if feat_split:
                pltpu.make_async_copy(src_hbm.at[s, g], sv, isem).wait()
                pltpu.make_async_copy(dst_hbm.at[s, g], dv, isem).wait()
            else:
                pltpu.make_async_copy(src_hbm.at[c, s, g], sv, isem).wait()
                pltpu.make_async_copy(dst_hbm.at[c, s, g], dv, isem).wait()

        def zrow(r, carry):
            def zcol(k, carry2):
                zb[r, pl.ds(k * 16, 16)] = jnp.zeros((16,), F32)
                return carry2

            return lax.fori_loop(0, 8, zcol, carry)

        lax.fori_loop(0, ZB, zrow, 0)
        load_idx(0, sva, dva)

        def zcp(j, carry):
            pltpu.async_copy(zb, acc_sh.at[pl.ds(s * RPT + j * ZB, ZB)], sem0)
            return carry

        lax.fori_loop(0, RPT // ZB, zcp, 0)

        def zwt(j, carry):
            pltpu.make_async_copy(
                zb, acc_sh.at[pl.ds(s * RPT + j * ZB, ZB)], sem0).wait()
            return carry

        lax.fori_loop(0, RPT // ZB, zwt, 0)
        wait_idx(0, sva, dva)
        pltpu.async_copy(table.at[tc].at[sva.at[0]], g0, sem0)
        plsc.subcore_barrier()

        def do_pair(sv, dv, j0, pf):
            j1 = j0 + 1
            pltpu.async_copy(table.at[tc].at[sv.at[j1]], g1, sem1)
            pltpu.make_async_copy(table.at[tc].at[sv.at[j0]], g0, sem0).wait()
            pltpu.sync_copy(g0, acc_sh.at[dv.at[j0]], add=True)
            if pf is not None:
                pltpu.async_copy(table.at[tc].at[pf[0].at[pf[1]]], g0, sem0)
            pltpu.make_async_copy(table.at[tc].at[sv.at[j1]], g1, sem1).wait()
            pltpu.sync_copy(g1, acc_sh.at[dv.at[j1]], add=True)

        def mid_pairs(sv, dv):
            def pp(p, carry2):
                do_pair(sv, dv, p * 2, (sv, p * 2 + 2))
                return carry2

            lax.fori_loop(0, GRP // 2 - 1, pp, 0)

        def two_groups(k, carry):
            gb = 2 * k + 1
            gn = jnp.minimum(gb + 1, NG - 1)
            load_idx(gb, svb, dvb)
            mid_pairs(sva, dva)
            wait_idx(gb, svb, dvb)
            do_pair(sva, dva, GRP - 2, (svb, 0))
            load_idx(gn, sva, dva)
            mid_pairs(svb, dvb)
            wait_idx(gn, sva, dva)
            do_pair(svb, dvb, GRP - 2, (sva, 0))
            return carry

        lax.fori_loop(0, NG // 2 - 1, two_groups, 0)
        load_idx(NG - 1, svb, dvb)
        mid_pairs(sva, dva)
        wait_idx(NG - 1, svb, dvb)
        do_pair(sva, dva, GRP - 2, (svb, 0))
        mid_pairs(svb, dvb)
        do_pair(svb, dvb, GRP - 2, None)
        plsc.subcore_barrier()

        pltpu.sync_copy(acc_sh.at[pl.ds(s * RPT, RPT)],
                        out_hbm.at[c, pl.ds(s * RPT, RPT)])

    return pl.kernel(
        body,
        out_type=jax.ShapeDtypeStruct((NC, NPAD, 128), F32),
        mesh=plsc.VectorSubcoreMesh(**_MESH),
        compiler_params=_SC_PARAMS,
        scratch_types=[
            pltpu.VMEM((GRP, CH), jnp.int32),
            pltpu.VMEM((GRP, CH), jnp.int32),
            pltpu.VMEM((GRP, CH), jnp.int32),
            pltpu.VMEM((GRP, CH), jnp.int32),
            pltpu.VMEM((CH, 128), F32),
            pltpu.VMEM((CH, 128), F32),
            pltpu.VMEM((ZB, 128), F32),
            pltpu.VMEM_SHARED((NPAD, 128), F32),
            pltpu.SemaphoreType.DMA,
            pltpu.SemaphoreType.DMA,
            pltpu.SemaphoreType.DMA,
        ],
    )


def _dinv_of(dT_blk):
    return lax.rsqrt(jnp.sum(dT_blk, axis=1, keepdims=True) + 1.0)


def _tc1_body(x_ref, w_ref, dT_ref, y_ref):
    dinv = _dinv_of(dT_ref[...])
    xw = jnp.dot(x_ref[...], w_ref[...], preferred_element_type=F32)
    y = xw * dinv
    y_ref[0] = y[:, :128]
    y_ref[1] = y[:, 128:]


def _tc1(x, W1, degT, interpret=False):
    return pl.pallas_call(
        _tc1_body,
        grid=(NPAD // NB,),
        in_specs=[
            pl.BlockSpec((NB, 128), lambda i: (i, 0)),
            pl.BlockSpec((128, 256), lambda i: (0, 0)),
            pl.BlockSpec((NB, NC), lambda i: (i, 0)),
        ],
        out_specs=pl.BlockSpec((2, NB, 128), lambda i: (0, i, 0)),
        out_shape=jax.ShapeDtypeStruct((2, NPAD, 128), F32),
        interpret=interpret,
    )(x, W1, degT)


def _tc2_body(a_ref, y_ref, dT_ref, w2_ref, b1_ref, o_ref):
    dinv = _dinv_of(dT_ref[...])
    h = jnp.concatenate([a_ref[0] + y_ref[0], a_ref[1] + y_ref[1]], axis=1)
    h = jnp.maximum(h * dinv + b1_ref[...], 0.0)
    o_ref[...] = jnp.dot(h, w2_ref[...], preferred_element_type=F32) * dinv


def _tc2(acc1, y1, degT, W2, b1r, interpret=False):
    return pl.pallas_call(
        _tc2_body,
        grid=(NPAD // NB,),
        in_specs=[
            pl.BlockSpec((2, NB, 128), lambda i: (0, i, 0)),
            pl.BlockSpec((2, NB, 128), lambda i: (0, i, 0)),
            pl.BlockSpec((NB, NC), lambda i: (i, 0)),
            pl.BlockSpec((256, 128), lambda i: (0, 0)),
            pl.BlockSpec((1, 256), lambda i: (0, 0)),
        ],
        out_specs=pl.BlockSpec((NB, 128), lambda i: (i, 0)),
        out_shape=jax.ShapeDtypeStruct((NPAD, 128), F32),
        interpret=interpret,
    )(acc1, y1, degT, W2, b1r)


def _tc3_body(a_ref, y_ref, dT_ref, b2_ref, wc_ref, bc_ref, o_ref):
    dinv = _dinv_of(dT_ref[...])
    h = jnp.maximum((a_ref[0] + a_ref[1] + y_ref[...]) * dinv + b2_ref[...], 0.0)
    logits = jnp.dot(h, wc_ref[...], preferred_element_type=F32) + bc_ref[...]
    m = jnp.max(logits, axis=1, keepdims=True)
    lse = jnp.log(jnp.sum(jnp.exp(logits - m), axis=1, keepdims=True))
    o_ref[...] = logits - m - lse


def _tc3(acc2, y2, degT, b2r, Wc, bcr, interpret=False):
    return pl.pallas_call(
        _tc3_body,
        grid=(NPAD // NB,),
        in_specs=[
            pl.BlockSpec((2, NB, 128), lambda i: (0, i, 0)),
            pl.BlockSpec((NB, 128), lambda i: (i, 0)),
            pl.BlockSpec((NB, NC), lambda i: (i, 0)),
            pl.BlockSpec((1, 128), lambda i: (0, 0)),
            pl.BlockSpec((128, 10), lambda i: (0, 0)),
            pl.BlockSpec((1, 10), lambda i: (0, 0)),
        ],
        out_specs=pl.BlockSpec((NB, 10), lambda i: (i, 0)),
        out_shape=jax.ShapeDtypeStruct((N, 10), F32),
        interpret=interpret,
    )(acc2, y2, degT, b2r, Wc, bcr)


def kernel(x, edge_index, W1, b1, W2, b2, Wc, bc):
    ei = edge_index.astype(jnp.int32)
    src = ei[0]
    dst = ei[1]

    degp = _deg(dst).reshape(NC, NPAD)
    degT = degp.T

    y1 = _tc1(x, W1, degT)

    nch1 = E // (NS * CH)
    src1 = src.reshape(NS, nch1 // GRP, GRP, CH)
    dst1 = dst.reshape(NS, nch1 // GRP, GRP, CH)
    acc1 = _make_prop(nch1, True)(y1, src1, dst1)

    y2 = _tc2(acc1, y1, degT, W2, b1.reshape(1, 256))

    nch2 = E // (NC * NS * CH)
    src2 = src.reshape(NC, NS, nch2 // GRP, GRP, CH)
    dst2 = dst.reshape(NC, NS, nch2 // GRP, GRP, CH)
    acc2 = _make_prop(nch2, False)(y2.reshape(1, NPAD, 128), src2, dst2)

    return _tc3(acc2, y2, degT, b2.reshape(1, 128), Wc, bc.reshape(1, 10))
